# baseline (speedup 1.0000x reference)
import functools

import jax
import jax.numpy as jnp
from jax import lax
from jax.experimental import pallas as pl
from jax.experimental.pallas import tpu as pltpu
from jax.experimental.pallas import tpu_sc as plsc

NC = 2
NS = 16
NW = NC * NS
K = 7
LANES = 16
G = 8
NBUF = 6


@functools.lru_cache(maxsize=None)
def _build(n_verts: int, feat: int, L: int):
    bpw = ((L + NW * G * NBUF - 1) // (NW * G * NBUF)) * (G * NBUF)
    L_pad = bpw * NW
    n_chunks = bpw // G
    n_outer = n_chunks // NBUF - 1
    mesh = plsc.VectorSubcoreMesh(
        core_axis_name="c", subcore_axis_name="s",
        num_cores=NC, num_subcores=NS)

    def body(x_hbm, idx_hbm, out_hbm, idx_v, rows, outs, gsems, osems):
        wid = lax.axis_index("s") * NC + lax.axis_index("c")
        base = wid * bpw
        pltpu.sync_copy(idx_hbm.at[pl.ds(base * K, bpw * K)], idx_v)

        def gather_start(c, b):
            pltpu.async_copy(
                x_hbm.at[idx_v.at[pl.ds(c * (G * K), G * K)]],
                rows[b], gsems[b])

        def gather_wait(b):
            pltpu.make_async_copy(
                x_hbm.at[idx_v.at[pl.ds(0, G * K)]],
                rows[b], gsems[b]).wait()

        def out_start(c, b):
            pltpu.async_copy(
                outs[b], out_hbm.at[pl.ds(base + c * G, G)], osems[b])

        def out_wait(b):
            pltpu.make_async_copy(
                outs[b], out_hbm.at[pl.ds(0, G)], osems[b]).wait()

        def compute(b):
            rv, ov = rows[b], outs[b]

            def row(g, carry):
                for d in range(feat // LANES):
                    sl = pl.ds(d * LANES, LANES)
                    acc = rv[g * K, sl]
                    for j in range(1, K):
                        acc = jnp.maximum(acc, rv[g * K + j, sl])
                    ov[g, sl] = acc
                return carry

            lax.fori_loop(0, G, row, 0)

        for b in range(NBUF):
            gather_start(b, b)

        def outer(o, carry):
            for b in range(NBUF):
                c = o * NBUF + b
                gather_wait(b)
                pl.when(o > 0)(lambda b=b: out_wait(b))
                compute(b)
                out_start(c, b)
                gather_start(c + NBUF, b)
            return carry

        lax.fori_loop(0, n_outer, outer, 0)

        for b in range(NBUF):
            c = n_outer * NBUF + b
            gather_wait(b)
            out_wait(b)
            compute(b)
            out_start(c, b)
        for b in range(NBUF):
            out_wait(b)

    kern = pl.kernel(
        body,
        out_type=jax.ShapeDtypeStruct((L_pad, feat), jnp.float32),
        mesh=mesh,
        scratch_types=[
            pltpu.VMEM((bpw * K,), jnp.int32),
            [pltpu.VMEM((G * K, feat), jnp.float32) for _ in range(NBUF)],
            [pltpu.VMEM((G, feat), jnp.float32) for _ in range(NBUF)],
            [pltpu.SemaphoreType.DMA for _ in range(NBUF)],
            [pltpu.SemaphoreType.DMA for _ in range(NBUF)],
        ],
    )
    return kern, L_pad


def kernel(x, hex_idx):
    n = hex_idx.shape[0]
    feat = x.shape[-1]
    x2 = x.reshape(n, -1)
    L = (n + 6) // 4
    kern, L_pad = _build(n, feat, L)
    idx = hex_idx[:L].astype(jnp.int32)
    idx = jnp.pad(idx, ((0, L_pad - L), (0, 0)))
    out = kern(x2, idx.reshape(-1))
    return out[:L]
